# Initial kernel scaffold; baseline (speedup 1.0000x reference)
import jax
import jax.numpy as jnp
from jax.experimental import pallas as pl


def kernel(inputs, condition, Wp, bp, We, be):
    raise NotImplementedError("write your pallas kernel here")


# speedup vs baseline: 5.0889x; 5.0889x over previous
import dataclasses
import functools

import jax
import jax.numpy as jnp
from jax import lax
from jax.experimental import pallas as pl
from jax.experimental.pallas import tpu as pltpu
from jax.experimental.pallas import tpu_sc as plsc

_LANES = 16


def _pred_body(c_ref, wp_ref, bp_ref, o_ref):
    o_ref[...] = (
        jnp.dot(c_ref[...], wp_ref[...], preferred_element_type=jnp.float32)
        + bp_ref[...]
    )


def _router_body(n_flat, n_ee, n_e, pred_hbm, o_hbm, pred_v, max_v, idx_v, out_v, sem):
    is_lead = jnp.logical_and(lax.axis_index("c") == 0, lax.axis_index("s") == 0)

    @pl.when(is_lead)
    def _():
        pltpu.async_copy(pred_hbm, pred_v, sem).wait()
        max_v[...] = pred_v[pl.ds(0, _LANES)]
        idx_v[...] = lax.iota(jnp.int32, _LANES)

        @pl.loop(1, n_flat // _LANES)
        def _(i):
            v = pred_v[pl.ds(i * _LANES, _LANES)]
            cur = max_v[...]
            take = v > cur
            pos = lax.iota(jnp.int32, _LANES) + i * _LANES
            idx_v[...] = jnp.where(take, pos, idx_v[...])
            max_v[...] = jnp.where(take, v, cur)

        m = jnp.max(max_v[...])
        cand = jnp.where(max_v[...] == m, idx_v[...], jnp.int32(n_flat))
        flat_idx = jnp.min(cand)
        expert = jnp.minimum(flat_idx, jnp.int32(n_ee - 1)) // jnp.int32(n_e)
        out_v[...] = jnp.full((_LANES,), 0, jnp.int32) + expert
        pltpu.async_copy(out_v, o_hbm, sem).wait()


def _expert_body(eidx_ref, x_ref, w_ref, b_ref, o_ref):
    del eidx_ref
    o_ref[...] = (
        jnp.dot(x_ref[...], w_ref[0], preferred_element_type=jnp.float32)
        + b_ref[0]
    )


def kernel(inputs, condition, Wp, bp, We, be):
    n, d = inputs.shape
    e = Wp.shape[1]
    n_flat = n * e

    pred = pl.pallas_call(
        _pred_body,
        out_shape=jax.ShapeDtypeStruct((n, e), jnp.float32),
    )(condition, Wp, bp.reshape(1, e))

    mesh = plsc.VectorSubcoreMesh(core_axis_name="c", subcore_axis_name="s")
    cp = pltpu.CompilerParams()
    if "needs_layout_passes" in pltpu.CompilerParams.__dataclass_fields__:
        cp = dataclasses.replace(cp, needs_layout_passes=False)
    router = pl.kernel(
        functools.partial(_router_body, n_flat, e * e, e),
        out_type=jax.ShapeDtypeStruct((_LANES,), jnp.int32),
        mesh=mesh,
        scratch_types=[
            pltpu.VMEM((n_flat,), jnp.float32),
            pltpu.VMEM((_LANES,), jnp.float32),
            pltpu.VMEM((_LANES,), jnp.int32),
            pltpu.VMEM((_LANES,), jnp.int32),
            pltpu.SemaphoreType.DMA,
        ],
        compiler_params=cp,
    )
    expert_vec = router(pred.reshape(n_flat))

    bn = 256
    grid_spec = pltpu.PrefetchScalarGridSpec(
        num_scalar_prefetch=1,
        grid=(n // bn,),
        in_specs=[
            pl.BlockSpec((bn, d), lambda i, eidx: (i, 0)),
            pl.BlockSpec((1, d, d), lambda i, eidx: (eidx[0], 0, 0)),
            pl.BlockSpec((1, 1, d), lambda i, eidx: (eidx[0], 0, 0)),
        ],
        out_specs=pl.BlockSpec((bn, d), lambda i, eidx: (i, 0)),
    )
    result = pl.pallas_call(
        _expert_body,
        grid_spec=grid_spec,
        out_shape=jax.ShapeDtypeStruct((n, d), jnp.float32),
    )(expert_vec, inputs, We, be.reshape(e, 1, d))
    return result


# speedup vs baseline: 5.7789x; 1.1356x over previous
import dataclasses
import functools

import jax
import jax.numpy as jnp
from jax import lax
from jax.experimental import pallas as pl
from jax.experimental.pallas import tpu as pltpu
from jax.experimental.pallas import tpu_sc as plsc

_LANES = 16


def _pred_body(c_ref, wp_ref, bp_ref, o_ref):
    o_ref[...] = (
        jnp.dot(c_ref[...], wp_ref[...], preferred_element_type=jnp.float32)
        + bp_ref[...]
    )


_NSUB = 16


def _router_body(
    n_flat, n_ee, n_e,
    pred_hbm, o_hbm,
    pred_v, max_v, idx_v, shmax, shidx, loc_max, loc_idx, out_v,
):
    cid = lax.axis_index("c")
    sid = lax.axis_index("s")
    chunk = n_flat // _NSUB

    @pl.when(cid == 0)
    def _():
        base = sid * chunk
        pltpu.sync_copy(pred_hbm.at[pl.ds(base, chunk)], pred_v)
        max_v[...] = pred_v[pl.ds(0, _LANES)]
        idx_v[...] = lax.iota(jnp.int32, _LANES) + base

        @pl.loop(1, chunk // _LANES)
        def _(i):
            v = pred_v[pl.ds(i * _LANES, _LANES)]
            cur = max_v[...]
            take = v > cur
            pos = lax.iota(jnp.int32, _LANES) + (base + i * _LANES)
            idx_v[...] = jnp.where(take, pos, idx_v[...])
            max_v[...] = jnp.where(take, v, cur)

        pltpu.sync_copy(max_v, shmax.at[sid])
        pltpu.sync_copy(idx_v, shidx.at[sid])

    plsc.subcore_barrier()

    @pl.when(jnp.logical_and(cid == 0, sid == 0))
    def _():
        pltpu.sync_copy(shmax, loc_max)
        pltpu.sync_copy(shidx, loc_idx)
        cur = loc_max[0]
        cidx = loc_idx[0]
        for w in range(1, _NSUB):
            v = loc_max[w]
            take = v > cur
            cidx = jnp.where(take, loc_idx[w], cidx)
            cur = jnp.where(take, v, cur)
        max_v[...] = cur
        idx_v[...] = cidx

        m = jnp.max(max_v[...])
        cand = jnp.where(max_v[...] == m, idx_v[...], jnp.int32(n_flat))
        flat_idx = jnp.min(cand)
        expert = jnp.minimum(flat_idx, jnp.int32(n_ee - 1)) // jnp.int32(n_e)
        out_v[...] = jnp.full((_LANES,), 0, jnp.int32) + expert
        pltpu.sync_copy(out_v, o_hbm)


def _expert_body(eidx_ref, x_ref, w_ref, b_ref, o_ref):
    del eidx_ref
    o_ref[...] = (
        jnp.dot(x_ref[...], w_ref[0], preferred_element_type=jnp.float32)
        + b_ref[0]
    )


def kernel(inputs, condition, Wp, bp, We, be):
    n, d = inputs.shape
    e = Wp.shape[1]
    n_flat = n * e

    pred = pl.pallas_call(
        _pred_body,
        out_shape=jax.ShapeDtypeStruct((n, e), jnp.float32),
    )(condition, Wp, bp.reshape(1, e))

    mesh = plsc.VectorSubcoreMesh(core_axis_name="c", subcore_axis_name="s")
    cp = pltpu.CompilerParams()
    if "needs_layout_passes" in pltpu.CompilerParams.__dataclass_fields__:
        cp = dataclasses.replace(cp, needs_layout_passes=False)
    router = pl.kernel(
        functools.partial(_router_body, n_flat, e * e, e),
        out_type=jax.ShapeDtypeStruct((_LANES,), jnp.int32),
        mesh=mesh,
        scratch_types=[
            pltpu.VMEM((n_flat // _NSUB,), jnp.float32),
            pltpu.VMEM((_LANES,), jnp.float32),
            pltpu.VMEM((_LANES,), jnp.int32),
            pltpu.VMEM_SHARED((_NSUB, _LANES), jnp.float32),
            pltpu.VMEM_SHARED((_NSUB, _LANES), jnp.int32),
            pltpu.VMEM((_NSUB, _LANES), jnp.float32),
            pltpu.VMEM((_NSUB, _LANES), jnp.int32),
            pltpu.VMEM((_LANES,), jnp.int32),
        ],
        compiler_params=cp,
    )
    expert_vec = router(pred.reshape(n_flat))

    bn = 256
    grid_spec = pltpu.PrefetchScalarGridSpec(
        num_scalar_prefetch=1,
        grid=(n // bn,),
        in_specs=[
            pl.BlockSpec((bn, d), lambda i, eidx: (i, 0)),
            pl.BlockSpec((1, d, d), lambda i, eidx: (eidx[0], 0, 0)),
            pl.BlockSpec((1, 1, d), lambda i, eidx: (eidx[0], 0, 0)),
        ],
        out_specs=pl.BlockSpec((bn, d), lambda i, eidx: (i, 0)),
    )
    result = pl.pallas_call(
        _expert_body,
        grid_spec=grid_spec,
        out_shape=jax.ShapeDtypeStruct((n, d), jnp.float32),
    )(expert_vec, inputs, We, be.reshape(e, 1, d))
    return result
